# baseline (speedup 1.0000x reference)
import functools

import jax
import jax.numpy as jnp
from jax import lax
from jax.experimental import pallas as pl
from jax.experimental.pallas import tpu as pltpu
from jax.experimental.pallas import tpu_sc as plsc

_NBUF = 4
_CHUNK = 416


@functools.partial(jax.jit, static_argnums=(2, 3))
def _embedding_lookup(idx_flat, weight, B, F):
    n = idx_flat.shape[0]
    V, D = weight.shape
    info = plsc.get_sparse_core_info()
    NC, NS = info.num_cores, info.num_subcores
    NW = NC * NS
    assert n % NW == 0
    b_per_w = n // NW
    C = _CHUNK
    NBUF = _NBUF
    assert b_per_w % (C * NBUF) == 0
    n_chunks = b_per_w // C

    mesh = plsc.VectorSubcoreMesh(core_axis_name="c", subcore_axis_name="s")

    @functools.partial(
        pl.kernel,
        mesh=mesh,
        out_type=jax.ShapeDtypeStruct((n * D,), jnp.float32),
        scratch_types=[
            pltpu.VMEM((b_per_w,), jnp.int32),
            *[pltpu.VMEM((C, D), jnp.float32) for _ in range(NBUF)],
            *[pltpu.VMEM((C * D,), jnp.float32) for _ in range(NBUF)],
            *[pltpu.SemaphoreType.DMA for _ in range(2 * NBUF)],
        ],
        compiler_params=pltpu.CompilerParams(use_tc_tiling_on_sc=False),
    )
    def emb(table_hbm, idx_hbm, out_hbm, idx_v, *bufs_and_sems):
        rows = bufs_and_sems[:NBUF]
        flat = bufs_and_sems[NBUF : 2 * NBUF]
        gsem = bufs_and_sems[2 * NBUF : 3 * NBUF]
        ssem = bufs_and_sems[3 * NBUF :]
        wid = lax.axis_index("s") * NC + lax.axis_index("c")
        base = wid * b_per_w

        def gather(j, b):
            return pltpu.make_async_copy(
                table_hbm.at[idx_v.at[pl.ds(j * C, C)]], rows[b], gsem[b]
            )

        def store(j, b):
            return pltpu.make_async_copy(
                flat[b], out_hbm.at[pl.ds((base + j * C) * D, C * D)], ssem[b]
            )

        def flatten(b):
            def body(r, carry):
                for u in range(8):
                    for h in range(D // 16):
                        flat[b][pl.ds((r * 8 + u) * D + h * 16, 16)] = rows[b][
                            r * 8 + u, pl.ds(h * 16, 16)
                        ]
                return carry

            lax.fori_loop(0, C // 8, body, 0)

        pltpu.sync_copy(idx_hbm.at[pl.ds(base, b_per_w)], idx_v)

        for b in range(NBUF):
            gather(b, b).start()

        def step(g, carry):
            for b in range(NBUF):
                j = g * NBUF + b
                gather(j, b).wait()

                @pl.when(j >= NBUF)
                def _():
                    store(j - NBUF, b).wait()

                flatten(b)
                store(j, b).start()
                jn = j + NBUF

                @pl.when(jn < n_chunks)
                def _():
                    gather(jn, b).start()

            return carry

        lax.fori_loop(0, n_chunks // NBUF, step, 0)

        for b in range(NBUF):
            store(n_chunks - NBUF + b, b).wait()

    return emb(weight, idx_flat).reshape(B, F, D)


def kernel(x, weight):
    B, F = x.shape
    return _embedding_lookup(x.reshape(B * F).astype(jnp.int32), weight, B, F)


# speedup vs baseline: 1.0018x; 1.0018x over previous
import functools

import jax
import jax.numpy as jnp
from jax import lax
from jax.experimental import pallas as pl
from jax.experimental.pallas import tpu as pltpu
from jax.experimental.pallas import tpu_sc as plsc

_NBUF = 4
_CHUNK = 416


@functools.partial(jax.jit, static_argnums=(2, 3))
def _embedding_lookup(idx_flat, weight, B, F):
    n = idx_flat.shape[0]
    V, D = weight.shape
    info = plsc.get_sparse_core_info()
    NC, NS = info.num_cores, info.num_subcores
    NW = NC * NS
    assert n % NW == 0
    b_per_w = n // NW
    C = _CHUNK
    NBUF = _NBUF
    assert b_per_w % (C * NBUF) == 0
    n_chunks = b_per_w // C

    mesh = plsc.VectorSubcoreMesh(core_axis_name="c", subcore_axis_name="s")

    @functools.partial(
        pl.kernel,
        mesh=mesh,
        out_type=jax.ShapeDtypeStruct((n * D,), jnp.float32),
        scratch_types=[
            pltpu.VMEM((b_per_w,), jnp.int32),
            *[pltpu.VMEM((C, D), jnp.float32) for _ in range(NBUF)],
            *[pltpu.VMEM((C * D,), jnp.float32) for _ in range(NBUF)],
            *[pltpu.SemaphoreType.DMA for _ in range(2 * NBUF)],
        ],
        compiler_params=pltpu.CompilerParams(use_tc_tiling_on_sc=False),
    )
    def emb(table_hbm, idx_hbm, out_hbm, idx_v, *bufs_and_sems):
        rows = bufs_and_sems[:NBUF]
        flat = bufs_and_sems[NBUF : 2 * NBUF]
        gsem = bufs_and_sems[2 * NBUF : 3 * NBUF]
        ssem = bufs_and_sems[3 * NBUF :]
        wid = lax.axis_index("s") * NC + lax.axis_index("c")
        base = wid * b_per_w

        def gather(j, b):
            return pltpu.make_async_copy(
                table_hbm.at[idx_v.at[pl.ds(j * C, C)]], rows[b], gsem[b]
            )

        def store(j, b):
            return pltpu.make_async_copy(
                flat[b], out_hbm.at[pl.ds((base + j * C) * D, C * D)], ssem[b]
            )

        def flatten(b):
            def body(r, carry):
                for u in range(8):
                    for h in range(D // 16):
                        flat[b][pl.ds((r * 8 + u) * D + h * 16, 16)] = rows[b][
                            r * 8 + u, pl.ds(h * 16, 16)
                        ]
                return carry

            lax.fori_loop(0, C // 8, body, 0)

        pltpu.sync_copy(idx_hbm.at[pl.ds(base, b_per_w)], idx_v)

        for b in range(NBUF):
            gather(b, b).start()

        def step(g, carry):
            for b in range(NBUF):
                j = g * NBUF + b
                gather(j, b).wait()

                @pl.when(j >= NBUF)
                def _():
                    store(j - NBUF, b).wait()

                flatten(b)
                store(j, b).start()
                jn = j + NBUF

                @pl.when(jn < n_chunks)
                def _():
                    gather(jn, b).start()

            return carry

        lax.fori_loop(0, n_chunks // NBUF, step, 0)

        for b in range(NBUF):
            store(n_chunks - NBUF + b, b).wait()

    return emb(weight, idx_flat).reshape(B, F, D)


def kernel(x, weight):
    B, F = x.shape
    V, D = weight.shape
    wlin = lax.optimization_barrier(weight.reshape(V * D))
    return _embedding_lookup(
        x.reshape(B * F).astype(jnp.int32), wlin.reshape(V, D), B, F
    )
